# Initial kernel scaffold; baseline (speedup 1.0000x reference)
import functools

import jax
import jax.numpy as jnp
from jax import lax
from jax.experimental import pallas as pl
from jax.experimental.pallas import tpu as pltpu
from jax.experimental.pallas import tpu_sc as plsc

N = 50000
E = 800000
D = 64
PACK = 4
EP = E // PACK
DP = D * PACK
BM = 1000

HALF = N // 2
ACC_ROWS = 25088
CHUNK = 640
NIDX = CHUNK // 128
NCHUNK = E // CHUNK
NSUB = 16
ITERS = (NCHUNK + NSUB - 1) // NSUB


def _mlp_body(rbf_ref, w1_ref, b1_ref, w2_ref, b2_ref, out_ref):
    a = jnp.dot(rbf_ref[...], w1_ref[...], preferred_element_type=jnp.float32)
    a = a + b1_ref[...]
    z = 0.5 * a
    sp = 2.0 * jnp.log1p(jnp.exp(z))
    a = jnp.where(z > 14.0, a, sp)
    o = jnp.dot(a, w2_ref[...], preferred_element_type=jnp.float32)
    out_ref[...] = o + b2_ref[...]


def _edge_mlp(rbf4, w1bd, b1t, w2bd, b2t):
    return pl.pallas_call(
        _mlp_body,
        grid=(EP // BM,),
        in_specs=[
            pl.BlockSpec((BM, DP), lambda i: (i, 0)),
            pl.BlockSpec((DP, DP), lambda i: (0, 0)),
            pl.BlockSpec((1, DP), lambda i: (0, 0)),
            pl.BlockSpec((DP, DP), lambda i: (0, 0)),
            pl.BlockSpec((1, DP), lambda i: (0, 0)),
        ],
        out_specs=pl.BlockSpec((BM, DP), lambda i: (i, 0)),
        out_shape=jax.ShapeDtypeStruct((EP, DP), jnp.float32),
    )(rbf4, w1bd, b1t, w2bd, b2t)


def _sc_body(x_hbm, h_hbm, src_hbm, dst_hbm, out_hbm,
             src_v, dst_v, sidx_v, x_rows, h_rows, acc, sem):
    c = lax.axis_index("c")
    s = lax.axis_index("s")
    lo = c * HALF

    def _zrow(i, carry):
        for c4 in range(4):
            x_rows[i, pl.ds(c4 * 16, 16)] = jnp.zeros((16,), jnp.float32)
        return carry
    lax.fori_loop(0, CHUNK, _zrow, 0)
    zbase = s * (ACC_ROWS // NSUB)
    for k in range(3):
        pltpu.sync_copy(x_rows.at[pl.ds(0, 512)],
                        acc.at[pl.ds(zbase + k * 512, 512)])
    pltpu.sync_copy(x_rows.at[pl.ds(0, 32)], acc.at[pl.ds(zbase + 1536, 32)])
    plsc.subcore_barrier()

    def _chunk(i, carry):
        t = s + i * NSUB
        valid = t < NCHUNK
        tc = jnp.minimum(t, NCHUNK - 1)
        pltpu.sync_copy(src_hbm.at[pl.ds(tc * NIDX, NIDX)], src_v)
        pltpu.sync_copy(dst_hbm.at[pl.ds(tc * NIDX, NIDX)], dst_v)
        cps = [pltpu.async_copy(x_hbm.at[src_v.at[j]],
                                x_rows.at[pl.ds(j * 128, 128)], sem)
               for j in range(NIDX)]
        pltpu.sync_copy(h_hbm.at[pl.ds(tc * CHUNK, CHUNK)], h_rows)
        for r in range(NIDX):
            for c8 in range(8):
                dv = dst_v[r, pl.ds(c8 * 16, 16)]
                loc = dv - lo
                m = (dv >= lo) & (dv < lo + HALF) & valid
                sidx_v[r, pl.ds(c8 * 16, 16)] = jnp.where(m, loc, HALF)
        for cp in cps:
            cp.wait()

        def _mrow(r, carry2):
            for c4 in range(4):
                sl = pl.ds(c4 * 16, 16)
                h_rows[r, sl] = h_rows[r, sl] * x_rows[r, sl]
            return carry2
        lax.fori_loop(0, CHUNK, _mrow, 0)
        for j in range(NIDX):
            pltpu.sync_copy(h_rows.at[pl.ds(j * 128, 128)],
                            acc.at[sidx_v.at[j]], add=True)
        return carry
    lax.fori_loop(0, ITERS, _chunk, 0)

    plsc.subcore_barrier()
    obase = lo + s * 1562
    pltpu.sync_copy(acc.at[pl.ds(s * 1562, 1562)],
                    out_hbm.at[pl.ds(obase, 1562)])

    @pl.when(s == NSUB - 1)
    def _tail():
        pltpu.sync_copy(acc.at[pl.ds(24992, 8)],
                        out_hbm.at[pl.ds(lo + 24992, 8)])


_sc_kernel = functools.partial(
    pl.kernel,
    mesh=plsc.VectorSubcoreMesh(core_axis_name="c", subcore_axis_name="s"),
    out_type=jax.ShapeDtypeStruct((N, D), jnp.float32),
    scratch_types=[
        pltpu.VMEM((NIDX, 128), jnp.int32),
        pltpu.VMEM((NIDX, 128), jnp.int32),
        pltpu.VMEM((NIDX, 128), jnp.int32),
        pltpu.VMEM((CHUNK, D), jnp.float32),
        pltpu.VMEM((CHUNK, D), jnp.float32),
        pltpu.VMEM_SHARED((ACC_ROWS, D), jnp.float32),
        pltpu.SemaphoreType.DMA,
    ],
)(_sc_body)


def kernel(x, rbf, edge_index, W1, b1, W2, b2):
    src = edge_index[0].astype(jnp.int32).reshape(E // 128, 128)
    dst = edge_index[1].astype(jnp.int32).reshape(E // 128, 128)
    eye4 = jnp.eye(PACK, dtype=jnp.float32)
    w1bd = jnp.kron(eye4, W1.T.astype(jnp.float32))
    w2bd = jnp.kron(eye4, W2.T.astype(jnp.float32))
    b1t = jnp.tile(b1, PACK).reshape(1, DP)
    b2t = jnp.tile(b2, PACK).reshape(1, DP)
    h4 = _edge_mlp(rbf.reshape(EP, DP), w1bd, b1t, w2bd, b2t)
    h = h4.reshape(E, D)
    return _sc_kernel(x, h, src, dst)


# speedup vs baseline: 1.9514x; 1.9514x over previous
import functools

import jax
import jax.numpy as jnp
from jax import lax
from jax.experimental import pallas as pl
from jax.experimental.pallas import tpu as pltpu
from jax.experimental.pallas import tpu_sc as plsc

N = 50000
E = 800000
D = 64
PACK = 4
EP = E // PACK
DP = D * PACK
BM = 1000

HALF = N // 2
ACC_ROWS = 25088
CHUNK = 128
NCHUNK = E // CHUNK
NSUB = 16
ITERS = (NCHUNK + NSUB - 1) // NSUB


def _mlp_body(rbf_ref, w1_ref, b1_ref, w2_ref, b2_ref, out_ref):
    a = jnp.dot(rbf_ref[...], w1_ref[...], preferred_element_type=jnp.float32)
    a = a + b1_ref[...]
    z = 0.5 * a
    sp = 2.0 * jnp.log1p(jnp.exp(z))
    a = jnp.where(z > 14.0, a, sp)
    o = jnp.dot(a, w2_ref[...], preferred_element_type=jnp.float32)
    out_ref[...] = o + b2_ref[...]


def _edge_mlp(rbf4, w1bd, b1t, w2bd, b2t):
    return pl.pallas_call(
        _mlp_body,
        grid=(EP // BM,),
        in_specs=[
            pl.BlockSpec((BM, DP), lambda i: (i, 0)),
            pl.BlockSpec((DP, DP), lambda i: (0, 0)),
            pl.BlockSpec((1, DP), lambda i: (0, 0)),
            pl.BlockSpec((DP, DP), lambda i: (0, 0)),
            pl.BlockSpec((1, DP), lambda i: (0, 0)),
        ],
        out_specs=pl.BlockSpec((BM, DP), lambda i: (i, 0)),
        out_shape=jax.ShapeDtypeStruct((EP, DP), jnp.float32),
    )(rbf4, w1bd, b1t, w2bd, b2t)


def _sc_body(x_hbm, h_hbm, src_hbm, dst_hbm, out_hbm,
             src_v, dst_v, sidx_v, x_rows, h_rows, acc, sem):
    c = lax.axis_index("c")
    s = lax.axis_index("s")
    lo = c * HALF

    def _zrow(i, carry):
        for c4 in range(4):
            x_rows[i, pl.ds(c4 * 16, 16)] = jnp.zeros((16,), jnp.float32)
        return carry
    lax.fori_loop(0, CHUNK, _zrow, 0)
    zbase = s * (ACC_ROWS // NSUB)
    for k in range(ACC_ROWS // NSUB // CHUNK):
        pltpu.sync_copy(x_rows.at[pl.ds(0, CHUNK)],
                        acc.at[pl.ds(zbase + k * CHUNK, CHUNK)])
    pltpu.sync_copy(x_rows.at[pl.ds(0, 32)], acc.at[pl.ds(zbase + 1536, 32)])
    plsc.subcore_barrier()

    def _chunk(i, carry):
        t = s + i * NSUB
        lo_eff = jnp.where(t < NCHUNK, lo, N + D)
        tc = jnp.minimum(t, NCHUNK - 1)
        pltpu.sync_copy(src_hbm.at[tc], src_v)
        pltpu.sync_copy(dst_hbm.at[tc], dst_v)
        cp = pltpu.async_copy(x_hbm.at[src_v.at[0]], x_rows, sem)
        pltpu.sync_copy(h_hbm.at[pl.ds(tc * CHUNK, CHUNK)], h_rows)
        for c8 in range(8):
            dv = dst_v[0, pl.ds(c8 * 16, 16)]
            loc = dv - lo_eff
            m = (dv >= lo_eff) & (dv < lo_eff + HALF)
            sidx_v[0, pl.ds(c8 * 16, 16)] = jnp.where(m, loc, HALF)
        cp.wait()

        def _mrow(r, carry2):
            for c4 in range(4):
                sl = pl.ds(c4 * 16, 16)
                h_rows[r, sl] = h_rows[r, sl] * x_rows[r, sl]
            return carry2
        lax.fori_loop(0, CHUNK, _mrow, 0)
        pltpu.sync_copy(h_rows, acc.at[sidx_v.at[0]], add=True)
        return carry
    lax.fori_loop(0, ITERS, _chunk, 0)

    plsc.subcore_barrier()
    obase = lo + s * 1560
    pltpu.sync_copy(acc.at[pl.ds(s * 1560, 1560)],
                    out_hbm.at[pl.ds(obase, 1560)])

    @pl.when(s == NSUB - 1)
    def _tail():
        pltpu.sync_copy(acc.at[pl.ds(24960, 40)],
                        out_hbm.at[pl.ds(lo + 24960, 40)])


_sc_kernel = functools.partial(
    pl.kernel,
    mesh=plsc.VectorSubcoreMesh(core_axis_name="c", subcore_axis_name="s"),
    compiler_params=pltpu.CompilerParams(use_tc_tiling_on_sc=False),
    out_type=jax.ShapeDtypeStruct((N, D), jnp.float32),
    scratch_types=[
        pltpu.VMEM((1, 128), jnp.int32),
        pltpu.VMEM((1, 128), jnp.int32),
        pltpu.VMEM((1, 128), jnp.int32),
        pltpu.VMEM((CHUNK, D), jnp.float32),
        pltpu.VMEM((CHUNK, D), jnp.float32),
        pltpu.VMEM_SHARED((ACC_ROWS, D), jnp.float32),
        pltpu.SemaphoreType.DMA,
    ],
)(_sc_body)


def kernel(x, rbf, edge_index, W1, b1, W2, b2):
    src = edge_index[0].astype(jnp.int32).reshape(NCHUNK, 1, 128)
    dst = edge_index[1].astype(jnp.int32).reshape(NCHUNK, 1, 128)
    eye4 = jnp.eye(PACK, dtype=jnp.float32)
    w1bd = jnp.kron(eye4, W1.T.astype(jnp.float32))
    w2bd = jnp.kron(eye4, W2.T.astype(jnp.float32))
    b1t = jnp.tile(b1, PACK).reshape(1, DP)
    b2t = jnp.tile(b2, PACK).reshape(1, DP)
    h4 = _edge_mlp(rbf.reshape(EP, DP), w1bd, b1t, w2bd, b2t)
    h = h4.reshape(E, D)
    return _sc_kernel(x, h, src, dst)
